# baseline (speedup 1.0000x reference)
import functools

import jax
import jax.numpy as jnp
from jax import lax
from jax.experimental import pallas as pl
from jax.experimental.pallas import tpu as pltpu
from jax.experimental.pallas import tpu_sc as plsc

N = 10000
E = 320000
CIN = 128
H = 256
NCH = 5
GW = NCH * H
NF = 128
GWH = NCH * NF
GW2 = 384
CW = 64
NG = 64
NPAD = 10240
NPADF = NPAD * NF
HALF = 5000
HALFP = 5120
NP2 = 2 * HALFP
NTEC = 16
RPT = HALFP // NTEC
ESLICE = E // NTEC
EB = 32
NB = ESLICE // EB
DB = 128
MB = 1000
GRID = N // MB
ZCH = 4096

_f32 = jnp.float32
_i32 = jnp.int32
_bf16 = jnp.bfloat16


def _leaky(v):
    return jnp.where(v >= 0, v, 0.01 * v)


def _split_pg(pg):
    p = pg[:, :H]
    ga = jnp.concatenate(
        [pg[:, H + c * H:H + c * H + NF] for c in range(NCH)], axis=1)
    gb = jnp.concatenate(
        [pg[:, H + c * H + NF:H + (c + 1) * H] for c in range(NCH)], axis=1)
    return p, ga, gb


def _enc_body(x_ref, ew_ref, eb_ref, wc_ref, bn_ref,
              h_ref, p_ref, ga_ref, gb_ref):
    h = jnp.dot(x_ref[...], ew_ref[...], preferred_element_type=_f32) + eb_ref[...]
    h_ref[...] = h
    pg = jnp.dot(h, wc_ref[...], preferred_element_type=_f32)
    p, ga, gb = _split_pg(pg)
    p_ref[...] = p + bn_ref[...]
    ga_ref[...] = ga.astype(_bf16)
    gb_ref[...] = gb.astype(_bf16)


def _build_enc(interpret=False):
    return pl.pallas_call(
        _enc_body,
        grid=(GRID,),
        in_specs=[
            pl.BlockSpec((MB, CIN), lambda i: (i, 0)),
            pl.BlockSpec((CIN, H), lambda i: (0, 0)),
            pl.BlockSpec((1, H), lambda i: (0, 0)),
            pl.BlockSpec((H, H + GW), lambda i: (0, 0)),
            pl.BlockSpec((1, H), lambda i: (0, 0)),
        ],
        out_specs=[
            pl.BlockSpec((MB, H), lambda i: (i, 0)),
            pl.BlockSpec((MB, H), lambda i: (i, 0)),
            pl.BlockSpec((MB, GWH), lambda i: (i, 0)),
            pl.BlockSpec((MB, GWH), lambda i: (i, 0)),
        ],
        out_shape=[
            jax.ShapeDtypeStruct((N, H), _f32),
            jax.ShapeDtypeStruct((N, H), _f32),
            jax.ShapeDtypeStruct((N, GWH), _bf16),
            jax.ShapeDtypeStruct((N, GWH), _bf16),
        ],
        interpret=interpret,
    )


def _scale(a, dv):
    inv = jnp.where(dv > 0, 1.0 / jnp.maximum(dv, 1.0), 0.0)
    return a * inv[:, None]


def _upd_body(hp_ref, p_ref, a_ref, dv_ref, wc_ref, bn_ref,
              h_ref, pout_ref, ga_ref, gb_ref):
    h = _leaky(hp_ref[...] + p_ref[...] + _scale(a_ref[...], dv_ref[0, 0, :]))
    h_ref[...] = h
    pg = jnp.dot(h, wc_ref[...], preferred_element_type=_f32)
    p, ga, gb = _split_pg(pg)
    pout_ref[...] = p + bn_ref[...]
    ga_ref[...] = ga.astype(_bf16)
    gb_ref[...] = gb.astype(_bf16)


def _build_upd(interpret=False):
    return pl.pallas_call(
        _upd_body,
        grid=(GRID,),
        in_specs=[
            pl.BlockSpec((MB, H), lambda i: (i, 0)),
            pl.BlockSpec((MB, H), lambda i: (i, 0)),
            pl.BlockSpec((MB, H), lambda i: (i, 0)),
            pl.BlockSpec((1, 1, MB), lambda i: (i, 0, 0)),
            pl.BlockSpec((H, H + GW), lambda i: (0, 0)),
            pl.BlockSpec((1, H), lambda i: (0, 0)),
        ],
        out_specs=[
            pl.BlockSpec((MB, H), lambda i: (i, 0)),
            pl.BlockSpec((MB, H), lambda i: (i, 0)),
            pl.BlockSpec((MB, GWH), lambda i: (i, 0)),
            pl.BlockSpec((MB, GWH), lambda i: (i, 0)),
        ],
        out_shape=[
            jax.ShapeDtypeStruct((N, H), _f32),
            jax.ShapeDtypeStruct((N, H), _f32),
            jax.ShapeDtypeStruct((N, GWH), _bf16),
            jax.ShapeDtypeStruct((N, GWH), _bf16),
        ],
        interpret=interpret,
    )


def _fin_body(hp_ref, p_ref, a_ref, dv_ref, bt_ref, out_ref, ps_scr, cnt_scr):
    i = pl.program_id(0)

    @pl.when(i == 0)
    def _():
        ps_scr[...] = jnp.zeros((NG, H), _f32)
        cnt_scr[...] = jnp.zeros((NG, H), _f32)

    h = _leaky(hp_ref[...] + p_ref[...] + _scale(a_ref[...], dv_ref[0, 0, :]))
    bvec = bt_ref[0, 0, :]
    oh = (lax.broadcasted_iota(_i32, (NG, MB), 0) == bvec[None, :]).astype(_f32)
    ps_scr[...] += jnp.dot(oh, h, preferred_element_type=_f32)
    cnt_scr[...] += jnp.dot(oh, jnp.ones((MB, H), _f32),
                            preferred_element_type=_f32)

    @pl.when(i == GRID - 1)
    def _():
        out_ref[...] = ps_scr[...] / jnp.maximum(cnt_scr[...], 1.0)


def _build_fin(interpret=False):
    return pl.pallas_call(
        _fin_body,
        grid=(GRID,),
        in_specs=[
            pl.BlockSpec((MB, H), lambda i: (i, 0)),
            pl.BlockSpec((MB, H), lambda i: (i, 0)),
            pl.BlockSpec((MB, H), lambda i: (i, 0)),
            pl.BlockSpec((1, 1, MB), lambda i: (i, 0, 0)),
            pl.BlockSpec((1, 1, MB), lambda i: (i, 0, 0)),
        ],
        out_specs=pl.BlockSpec((NG, H), lambda i: (0, 0)),
        out_shape=jax.ShapeDtypeStruct((NG, H), _f32),
        scratch_shapes=[
            pltpu.VMEM((NG, H), _f32),
            pltpu.VMEM((NG, H), _f32),
        ],
        interpret=interpret,
    )


def _sc_mesh():
    return plsc.VectorSubcoreMesh(core_axis_name="c", subcore_axis_name="s")


def _build_deg():
    @functools.partial(
        pl.kernel,
        out_type=jax.ShapeDtypeStruct((NP2,), _f32),
        mesh=_sc_mesh(),
        scratch_types=[
            pltpu.VMEM_SHARED((HALFP,), _f32),
            pltpu.VMEM((ESLICE,), _i32),
            pltpu.VMEM((DB,), _i32),
            pltpu.VMEM((DB,), _f32),
            pltpu.VMEM((RPT,), _f32),
        ],
    )
    def deg_kernel(row_hbm, out_hbm, acc, rowv, oidx, ones_v, wbuf):
        c = lax.axis_index("c")
        s = lax.axis_index("s")

        def zw(j, _):
            wbuf[pl.ds(j * 16, 16)] = jnp.zeros((16,), _f32)
            return 0
        lax.fori_loop(0, RPT // 16, zw, 0)
        pltpu.sync_copy(wbuf, acc.at[pl.ds(s * RPT, RPT)])

        def zo(j, _):
            ones_v[pl.ds(j * 16, 16)] = jnp.ones((16,), _f32)
            return 0
        lax.fori_loop(0, DB // 16, zo, 0)

        pltpu.sync_copy(row_hbm.at[pl.ds(s * ESLICE, ESLICE)], rowv)
        plsc.subcore_barrier()

        half_base = c * HALF

        def body(b, _):
            def grp(g, _):
                rv = rowv[pl.ds(b * DB + g * 16, 16)]
                loc = rv - half_base
                ok = (loc >= 0) & (loc < HALF)
                oidx[pl.ds(g * 16, 16)] = jnp.where(ok, loc, HALF)
                return 0
            lax.fori_loop(0, DB // 16, grp, 0)
            pltpu.sync_copy(ones_v, acc.at[oidx], add=True)
            return 0
        lax.fori_loop(0, ESLICE // DB, body, 0)
        plsc.subcore_barrier()

        pltpu.sync_copy(acc.at[pl.ds(s * RPT, RPT)], wbuf)
        pltpu.sync_copy(wbuf, out_hbm.at[pl.ds(c * HALFP + s * RPT, RPT)])

    return deg_kernel


CHUNK = 800
NCHUNK = ESLICE // CHUNK
UE = 32
UNITS = CHUNK // UE


def _build_agg():
    @functools.partial(
        pl.kernel,
        out_type=jax.ShapeDtypeStruct((2 * NPADF,), _f32),
        mesh=_sc_mesh(),
        scratch_types=[
            pltpu.VMEM_SHARED((NPADF,), _f32),
            pltpu.VMEM((CHUNK,), _i32),
            pltpu.VMEM((CHUNK,), _i32),
            pltpu.VMEM((2, 32, GW2), _i32),
            pltpu.VMEM((2, 32, NF), _f32),
            pltpu.VMEM((NF,), _i32),
            pltpu.VMEM((CHUNK,), _f32),
            pltpu.VMEM((CHUNK,), _f32),
            pltpu.VMEM((CHUNK,), _f32),
            pltpu.VMEM((CHUNK,), _f32),
            pltpu.VMEM((ZCH,), _f32),
            pltpu.SemaphoreType.DMA,
            pltpu.SemaphoreType.DMA,
            pltpu.SemaphoreType.DMA,
            pltpu.SemaphoreType.DMA,
            pltpu.SemaphoreType.DMA,
        ],
    )
    def agg_kernel(ga_hbm, gb_hbm, row_hbm, col_hbm, pe0_hbm, pe1_hbm,
                   out_hbm, acc, idxr, idxc, gbuf, mbuf, iconst,
                   p0r, p0c, p1r, p1c, wbuf,
                   sem_p, sem_g0, sem_g1, sem_s0, sem_s1):
        c = lax.axis_index("c")
        s = lax.axis_index("s")

        def zg(j, _):
            wbuf[pl.ds(j * 16, 16)] = jnp.zeros((16,), _f32)
            return 0
        lax.fori_loop(0, ZCH // 16, zg, 0)
        stripe = NPADF // NTEC

        def zc(k, _):
            pltpu.sync_copy(wbuf, acc.at[pl.ds(s * stripe + k * ZCH, ZCH)])
            return 0
        lax.fori_loop(0, stripe // ZCH, zc, 0)
        plsc.subcore_barrier()

        def zi(k, _):
            iconst[pl.ds(k * 16, 16)] = lax.iota(_i32, 16) + k * 16
            return 0
        lax.fori_loop(0, NF // 16, zi, 0)
        sem_g = (sem_g0, sem_g1)
        sem_s = (sem_s0, sem_s1)

        def issue_gather(u, par):
            src = idxc.at[pl.ds(u * UE, UE)]

            @pl.when(c == 0)
            def _():
                pltpu.async_copy(ga_hbm.at[src], gbuf.at[par], sem_g[par])

            @pl.when(c != 0)
            def _():
                pltpu.async_copy(gb_hbm.at[src], gbuf.at[par], sem_g[par])

        def wait_gather(par):
            pltpu.make_async_copy(
                ga_hbm.at[idxc.at[pl.ds(0, UE)]], gbuf.at[par],
                sem_g[par]).wait()

        def drain_scatter(par):
            for e in range(UE):
                pltpu.make_async_copy(
                    mbuf.at[par, e], acc.at[pl.ds(0, NF)].at[iconst],
                    sem_s[par]).wait()

        def do_unit(u, par, k):
            wait_gather(par)

            @pl.when(u + 1 < UNITS)
            def _():
                issue_gather(u + 1, 1 - par)

            @pl.when(k > 0)
            def _():
                drain_scatter(par)

            for grp in range(UE // 16):
                off = u * UE + grp * 16
                d0 = p0c[pl.ds(off, 16)] - p0r[pl.ds(off, 16)]
                d1v = p1c[pl.ds(off, 16)] - p1r[pl.ds(off, 16)]
                wu0 = _leaky(d0)
                wd0 = _leaky(-d0)
                wu1 = _leaky(d1v)
                wd1 = _leaky(-d1v)
                rbase16 = idxr[pl.ds(off, 16)] * NF
                for jj in range(16):
                    j = grp * 16 + jj
                    a0 = wu0[jj]
                    a1 = wd0[jj]
                    a2 = wu1[jj]
                    a3 = wd1[jj]
                    rb = rbase16[jj]
                    for v in range(NF // 32):
                        us = [gbuf[par, j, pl.ds(cc * CW + v * 16, 16)]
                              for cc in range(NCH)]
                        ge = [lax.bitcast_convert_type(
                            lax.shift_left(w, 16), _f32) for w in us]
                        go = [lax.bitcast_convert_type(
                            lax.bitwise_and(w, jnp.int32(-65536)), _f32)
                            for w in us]
                        me = (ge[0] + a0 * ge[1] + a1 * ge[2]
                              + a2 * ge[3] + a3 * ge[4])
                        mo = (go[0] + a0 * go[1] + a1 * go[2]
                              + a2 * go[3] + a3 * go[4])
                        mbuf[par, j, pl.ds((2 * v) * 16, 16)] = me
                        mbuf[par, j, pl.ds((2 * v + 1) * 16, 16)] = mo
                    pltpu.async_copy(
                        mbuf.at[par, j],
                        acc.at[pl.ds(pl.multiple_of(rb, NF), NF)].at[iconst],
                        sem_s[par], add=True)

        def chunk_body(ck, _):
            ebase = s * ESLICE + ck * CHUNK
            pltpu.sync_copy(row_hbm.at[pl.ds(ebase, CHUNK)], idxr)
            pltpu.sync_copy(col_hbm.at[pl.ds(ebase, CHUNK)], idxc)
            d1 = pltpu.async_copy(pe0_hbm.at[idxr], p0r, sem_p)
            d2 = pltpu.async_copy(pe0_hbm.at[idxc], p0c, sem_p)
            d3 = pltpu.async_copy(pe1_hbm.at[idxr], p1r, sem_p)
            d4 = pltpu.async_copy(pe1_hbm.at[idxc], p1c, sem_p)
            d1.wait()
            d2.wait()
            d3.wait()
            d4.wait()
            issue_gather(0, 0)

            def pair(k, _):
                do_unit(2 * k, 0, k)
                do_unit(2 * k + 1, 1, k)
                return 0
            lax.fori_loop(0, UNITS // 2, pair, 0)
            do_unit(UNITS - 1, 0, UNITS // 2)
            drain_scatter(0)
            drain_scatter(1)
            return 0
        lax.fori_loop(0, NCHUNK, chunk_body, 0)
        plsc.subcore_barrier()

        def wb(k, _):
            off = s * stripe + k * ZCH
            pltpu.sync_copy(acc.at[pl.ds(off, ZCH)], wbuf)
            pltpu.sync_copy(wbuf, out_hbm.at[pl.ds(c * NPADF + off, ZCH)])
            return 0
        lax.fori_loop(0, stripe // ZCH, wb, 0)

    return agg_kernel


def kernel(x, edge_index, batch, pe, enc_W, enc_b,
           W1, b1, W2, b2, W3, b3, W4, b4):
    row = edge_index[0].astype(_i32)
    col = edge_index[1].astype(_i32)
    batch3d = batch.astype(_i32).reshape(GRID, 1, MB)
    pe0 = jnp.asarray(pe[:, 0], _f32)
    pe1 = jnp.asarray(pe[:, 1], _f32)

    def wcat(W):
        blocks = W.reshape(NCH + 1, H, H)
        return jnp.concatenate([blocks[i] for i in range(NCH + 1)], axis=1)

    Ws = [wcat(W) for W in (W1, W2, W3, W4)]
    bs = [b.reshape(1, H) for b in (b1, b2, b3, b4)]

    deg_pad = _build_deg()(row)
    deg3d = jnp.concatenate(
        [deg_pad[:HALF], deg_pad[HALFP:HALFP + HALF]]).reshape(GRID, 1, MB)

    enc = _build_enc()
    upd = _build_upd()
    fin = _build_fin()
    agg = _build_agg()

    def pack_g(g):
        t = g.reshape(N, GWH // 32, 2, 16).transpose(0, 1, 3, 2)
        g32 = lax.bitcast_convert_type(t, _i32).reshape(N, GWH // 2)
        return jnp.concatenate(
            [g32, jnp.zeros((N, GW2 - GWH // 2), _i32)], axis=1)

    def run_agg(ga, gb):
        flat = agg(pack_g(ga), pack_g(gb), row, col, pe0, pe1)
        a0 = flat[:N * NF].reshape(N, NF)
        a1 = flat[NPADF:NPADF + N * NF].reshape(N, NF)
        return jnp.concatenate([a0, a1], axis=1)

    h, p, ga, gb = enc(x, enc_W, enc_b.reshape(1, H), Ws[0], bs[0])
    for l in range(1, 4):
        a = run_agg(ga, gb)
        h, p, ga, gb = upd(h, p, a, deg3d, Ws[l], bs[l])
    a = run_agg(ga, gb)
    return fin(h, p, a, deg3d, batch3d)


# speedup vs baseline: 1.1889x; 1.1889x over previous
import functools

import jax
import jax.numpy as jnp
from jax import lax
from jax.experimental import pallas as pl
from jax.experimental.pallas import tpu as pltpu
from jax.experimental.pallas import tpu_sc as plsc

N = 10000
E = 320000
CIN = 128
H = 256
NCH = 5
GW = NCH * H
NF = 128
GWH = NCH * NF
GW2 = 384
CW = 64
NG = 64
NPAD = 10240
NPADF = NPAD * NF
HALF = 5000
HALFP = 5120
NP2 = 2 * HALFP
NTEC = 16
RPT = HALFP // NTEC
ESLICE = E // NTEC
EB = 32
NB = ESLICE // EB
DB = 128
MB = 1000
GRID = N // MB
ZCH = 4096

_f32 = jnp.float32
_i32 = jnp.int32
_bf16 = jnp.bfloat16


def _leaky(v):
    return jnp.where(v >= 0, v, 0.01 * v)


def _split_pg(pg):
    p = pg[:, :H]
    ga = jnp.concatenate(
        [pg[:, H + c * H:H + c * H + NF] for c in range(NCH)], axis=1)
    gb = jnp.concatenate(
        [pg[:, H + c * H + NF:H + (c + 1) * H] for c in range(NCH)], axis=1)
    return p, ga, gb


def _enc_body(x_ref, ew_ref, eb_ref, wc_ref, bn_ref,
              h_ref, p_ref, ga_ref, gb_ref):
    h = jnp.dot(x_ref[...], ew_ref[...], preferred_element_type=_f32) + eb_ref[...]
    h_ref[...] = h
    pg = jnp.dot(h, wc_ref[...], preferred_element_type=_f32)
    p, ga, gb = _split_pg(pg)
    p_ref[...] = p + bn_ref[...]
    ga_ref[...] = ga.astype(_bf16)
    gb_ref[...] = gb.astype(_bf16)


def _build_enc(interpret=False):
    return pl.pallas_call(
        _enc_body,
        grid=(GRID,),
        in_specs=[
            pl.BlockSpec((MB, CIN), lambda i: (i, 0)),
            pl.BlockSpec((CIN, H), lambda i: (0, 0)),
            pl.BlockSpec((1, H), lambda i: (0, 0)),
            pl.BlockSpec((H, H + GW), lambda i: (0, 0)),
            pl.BlockSpec((1, H), lambda i: (0, 0)),
        ],
        out_specs=[
            pl.BlockSpec((MB, H), lambda i: (i, 0)),
            pl.BlockSpec((MB, H), lambda i: (i, 0)),
            pl.BlockSpec((MB, GWH), lambda i: (i, 0)),
            pl.BlockSpec((MB, GWH), lambda i: (i, 0)),
        ],
        out_shape=[
            jax.ShapeDtypeStruct((N, H), _f32),
            jax.ShapeDtypeStruct((N, H), _f32),
            jax.ShapeDtypeStruct((N, GWH), _bf16),
            jax.ShapeDtypeStruct((N, GWH), _bf16),
        ],
        interpret=interpret,
    )


def _scale(a, dv):
    inv = jnp.where(dv > 0, 1.0 / jnp.maximum(dv, 1.0), 0.0)
    return a * inv[:, None]


def _upd_body(hp_ref, p_ref, a_ref, dv_ref, wc_ref, bn_ref,
              h_ref, pout_ref, ga_ref, gb_ref):
    h = _leaky(hp_ref[...] + p_ref[...] + _scale(a_ref[...], dv_ref[0, 0, :]))
    h_ref[...] = h
    pg = jnp.dot(h, wc_ref[...], preferred_element_type=_f32)
    p, ga, gb = _split_pg(pg)
    pout_ref[...] = p + bn_ref[...]
    ga_ref[...] = ga.astype(_bf16)
    gb_ref[...] = gb.astype(_bf16)


def _build_upd(interpret=False):
    return pl.pallas_call(
        _upd_body,
        grid=(GRID,),
        in_specs=[
            pl.BlockSpec((MB, H), lambda i: (i, 0)),
            pl.BlockSpec((MB, H), lambda i: (i, 0)),
            pl.BlockSpec((MB, H), lambda i: (i, 0)),
            pl.BlockSpec((1, 1, MB), lambda i: (i, 0, 0)),
            pl.BlockSpec((H, H + GW), lambda i: (0, 0)),
            pl.BlockSpec((1, H), lambda i: (0, 0)),
        ],
        out_specs=[
            pl.BlockSpec((MB, H), lambda i: (i, 0)),
            pl.BlockSpec((MB, H), lambda i: (i, 0)),
            pl.BlockSpec((MB, GWH), lambda i: (i, 0)),
            pl.BlockSpec((MB, GWH), lambda i: (i, 0)),
        ],
        out_shape=[
            jax.ShapeDtypeStruct((N, H), _f32),
            jax.ShapeDtypeStruct((N, H), _f32),
            jax.ShapeDtypeStruct((N, GWH), _bf16),
            jax.ShapeDtypeStruct((N, GWH), _bf16),
        ],
        interpret=interpret,
    )


def _fin_body(hp_ref, p_ref, a_ref, dv_ref, bt_ref, out_ref, ps_scr, cnt_scr):
    i = pl.program_id(0)

    @pl.when(i == 0)
    def _():
        ps_scr[...] = jnp.zeros((NG, H), _f32)
        cnt_scr[...] = jnp.zeros((NG, H), _f32)

    h = _leaky(hp_ref[...] + p_ref[...] + _scale(a_ref[...], dv_ref[0, 0, :]))
    bvec = bt_ref[0, 0, :]
    oh = (lax.broadcasted_iota(_i32, (NG, MB), 0) == bvec[None, :]).astype(_f32)
    ps_scr[...] += jnp.dot(oh, h, preferred_element_type=_f32)
    cnt_scr[...] += jnp.dot(oh, jnp.ones((MB, H), _f32),
                            preferred_element_type=_f32)

    @pl.when(i == GRID - 1)
    def _():
        out_ref[...] = ps_scr[...] / jnp.maximum(cnt_scr[...], 1.0)


def _build_fin(interpret=False):
    return pl.pallas_call(
        _fin_body,
        grid=(GRID,),
        in_specs=[
            pl.BlockSpec((MB, H), lambda i: (i, 0)),
            pl.BlockSpec((MB, H), lambda i: (i, 0)),
            pl.BlockSpec((MB, H), lambda i: (i, 0)),
            pl.BlockSpec((1, 1, MB), lambda i: (i, 0, 0)),
            pl.BlockSpec((1, 1, MB), lambda i: (i, 0, 0)),
        ],
        out_specs=pl.BlockSpec((NG, H), lambda i: (0, 0)),
        out_shape=jax.ShapeDtypeStruct((NG, H), _f32),
        scratch_shapes=[
            pltpu.VMEM((NG, H), _f32),
            pltpu.VMEM((NG, H), _f32),
        ],
        interpret=interpret,
    )


def _sc_mesh():
    return plsc.VectorSubcoreMesh(core_axis_name="c", subcore_axis_name="s")


def _build_deg():
    @functools.partial(
        pl.kernel,
        out_type=jax.ShapeDtypeStruct((NP2,), _f32),
        mesh=_sc_mesh(),
        scratch_types=[
            pltpu.VMEM_SHARED((HALFP,), _f32),
            pltpu.VMEM((ESLICE,), _i32),
            pltpu.VMEM((DB,), _i32),
            pltpu.VMEM((DB,), _f32),
            pltpu.VMEM((RPT,), _f32),
        ],
    )
    def deg_kernel(row_hbm, out_hbm, acc, rowv, oidx, ones_v, wbuf):
        c = lax.axis_index("c")
        s = lax.axis_index("s")

        def zw(j, _):
            wbuf[pl.ds(j * 16, 16)] = jnp.zeros((16,), _f32)
            return 0
        lax.fori_loop(0, RPT // 16, zw, 0)
        pltpu.sync_copy(wbuf, acc.at[pl.ds(s * RPT, RPT)])

        def zo(j, _):
            ones_v[pl.ds(j * 16, 16)] = jnp.ones((16,), _f32)
            return 0
        lax.fori_loop(0, DB // 16, zo, 0)

        pltpu.sync_copy(row_hbm.at[pl.ds(s * ESLICE, ESLICE)], rowv)
        plsc.subcore_barrier()

        half_base = c * HALF

        def body(b, _):
            def grp(g, _):
                rv = rowv[pl.ds(b * DB + g * 16, 16)]
                loc = rv - half_base
                ok = (loc >= 0) & (loc < HALF)
                oidx[pl.ds(g * 16, 16)] = jnp.where(ok, loc, HALF)
                return 0
            lax.fori_loop(0, DB // 16, grp, 0)
            pltpu.sync_copy(ones_v, acc.at[oidx], add=True)
            return 0
        lax.fori_loop(0, ESLICE // DB, body, 0)
        plsc.subcore_barrier()

        pltpu.sync_copy(acc.at[pl.ds(s * RPT, RPT)], wbuf)
        pltpu.sync_copy(wbuf, out_hbm.at[pl.ds(c * HALFP + s * RPT, RPT)])

    return deg_kernel


CHUNK = 800
NCHUNK = ESLICE // CHUNK
UNITS = CHUNK // 16


def _build_agg():
    @functools.partial(
        pl.kernel,
        out_type=jax.ShapeDtypeStruct((2 * NPADF,), _f32),
        mesh=_sc_mesh(),
        scratch_types=[
            pltpu.VMEM_SHARED((NPADF,), _f32),
            pltpu.VMEM((CHUNK,), _i32),
            pltpu.VMEM((CHUNK,), _i32),
            pltpu.VMEM((2, 16, GW2), _i32),
            pltpu.VMEM((2, 16, NF), _f32),
            pltpu.VMEM((NF,), _i32),
            pltpu.VMEM((CHUNK,), _f32),
            pltpu.VMEM((CHUNK,), _f32),
            pltpu.VMEM((CHUNK,), _f32),
            pltpu.VMEM((CHUNK,), _f32),
            pltpu.VMEM((ZCH,), _f32),
            pltpu.SemaphoreType.DMA,
            pltpu.SemaphoreType.DMA,
            pltpu.SemaphoreType.DMA,
            pltpu.SemaphoreType.DMA,
            pltpu.SemaphoreType.DMA,
        ],
    )
    def agg_kernel(ga_hbm, gb_hbm, row_hbm, col_hbm, pe0_hbm, pe1_hbm,
                   out_hbm, acc, idxr, idxc, gbuf, mbuf, iconst,
                   p0r, p0c, p1r, p1c, wbuf,
                   sem_p, sem_g0, sem_g1, sem_s0, sem_s1):
        c = lax.axis_index("c")
        s = lax.axis_index("s")

        def zg(j, _):
            wbuf[pl.ds(j * 16, 16)] = jnp.zeros((16,), _f32)
            return 0
        lax.fori_loop(0, ZCH // 16, zg, 0)
        stripe = NPADF // NTEC

        def zc(k, _):
            pltpu.sync_copy(wbuf, acc.at[pl.ds(s * stripe + k * ZCH, ZCH)])
            return 0
        lax.fori_loop(0, stripe // ZCH, zc, 0)
        plsc.subcore_barrier()

        def zi(k, _):
            iconst[pl.ds(k * 16, 16)] = lax.iota(_i32, 16) + k * 16
            return 0
        lax.fori_loop(0, NF // 16, zi, 0)
        sem_g = (sem_g0, sem_g1)
        sem_s = (sem_s0, sem_s1)

        def issue_gather(u, par):
            src = idxc.at[pl.ds(u * 16, 16)]

            @pl.when(c == 0)
            def _():
                pltpu.async_copy(ga_hbm.at[src], gbuf.at[par], sem_g[par])

            @pl.when(c != 0)
            def _():
                pltpu.async_copy(gb_hbm.at[src], gbuf.at[par], sem_g[par])

        def wait_gather(par):
            pltpu.make_async_copy(
                ga_hbm.at[idxc.at[pl.ds(0, 16)]], gbuf.at[par],
                sem_g[par]).wait()

        def drain_scatter(par):
            for e in range(16):
                pltpu.make_async_copy(
                    mbuf.at[par, e], acc.at[pl.ds(0, NF)].at[iconst],
                    sem_s[par]).wait()

        def do_unit(u, par, k):
            wait_gather(par)

            @pl.when(u + 1 < UNITS)
            def _():
                issue_gather(u + 1, 1 - par)

            @pl.when(k > 0)
            def _():
                drain_scatter(par)

            off = u * 16
            d0 = p0c[pl.ds(off, 16)] - p0r[pl.ds(off, 16)]
            d1v = p1c[pl.ds(off, 16)] - p1r[pl.ds(off, 16)]
            wu0 = _leaky(d0)
            wd0 = _leaky(-d0)
            wu1 = _leaky(d1v)
            wd1 = _leaky(-d1v)
            rbase16 = idxr[pl.ds(off, 16)] * NF
            for j in range(16):
                a0 = wu0[j]
                a1 = wd0[j]
                a2 = wu1[j]
                a3 = wd1[j]
                rb = rbase16[j]
                for v in range(NF // 32):
                    us = [gbuf[par, j, pl.ds(cc * CW + v * 16, 16)]
                          for cc in range(NCH)]
                    ge = [lax.bitcast_convert_type(
                        lax.shift_left(u, 16), _f32) for u in us]
                    go = [lax.bitcast_convert_type(
                        lax.bitwise_and(u, jnp.int32(-65536)), _f32)
                        for u in us]
                    me = (ge[0] + a0 * ge[1] + a1 * ge[2]
                          + a2 * ge[3] + a3 * ge[4])
                    mo = (go[0] + a0 * go[1] + a1 * go[2]
                          + a2 * go[3] + a3 * go[4])
                    mbuf[par, j, pl.ds((2 * v) * 16, 16)] = me
                    mbuf[par, j, pl.ds((2 * v + 1) * 16, 16)] = mo
                pltpu.async_copy(
                    mbuf.at[par, j],
                    acc.at[pl.ds(pl.multiple_of(rb, NF), NF)].at[iconst],
                    sem_s[par], add=True)

        def chunk_body(ck, _):
            ebase = s * ESLICE + ck * CHUNK
            pltpu.sync_copy(row_hbm.at[pl.ds(ebase, CHUNK)], idxr)
            pltpu.sync_copy(col_hbm.at[pl.ds(ebase, CHUNK)], idxc)
            d1 = pltpu.async_copy(pe0_hbm.at[idxr], p0r, sem_p)
            d2 = pltpu.async_copy(pe0_hbm.at[idxc], p0c, sem_p)
            d3 = pltpu.async_copy(pe1_hbm.at[idxr], p1r, sem_p)
            d4 = pltpu.async_copy(pe1_hbm.at[idxc], p1c, sem_p)
            d1.wait()
            d2.wait()
            d3.wait()
            d4.wait()
            issue_gather(0, 0)

            def pair(k, _):
                do_unit(2 * k, 0, k)
                do_unit(2 * k + 1, 1, k)
                return 0
            lax.fori_loop(0, UNITS // 2, pair, 0)
            drain_scatter(0)
            drain_scatter(1)
            return 0
        lax.fori_loop(0, NCHUNK, chunk_body, 0)
        plsc.subcore_barrier()

        def wb(k, _):
            off = s * stripe + k * ZCH
            pltpu.sync_copy(acc.at[pl.ds(off, ZCH)], wbuf)
            pltpu.sync_copy(wbuf, out_hbm.at[pl.ds(c * NPADF + off, ZCH)])
            return 0
        lax.fori_loop(0, stripe // ZCH, wb, 0)

    return agg_kernel


def kernel(x, edge_index, batch, pe, enc_W, enc_b,
           W1, b1, W2, b2, W3, b3, W4, b4):
    row = edge_index[0].astype(_i32)
    col = edge_index[1].astype(_i32)
    batch3d = batch.astype(_i32).reshape(GRID, 1, MB)
    pe0 = jnp.asarray(pe[:, 0], _f32)
    pe1 = jnp.asarray(pe[:, 1], _f32)

    def wcat(W):
        blocks = W.reshape(NCH + 1, H, H)
        return jnp.concatenate([blocks[i] for i in range(NCH + 1)], axis=1)

    Ws = [wcat(W) for W in (W1, W2, W3, W4)]
    bs = [b.reshape(1, H) for b in (b1, b2, b3, b4)]

    deg_pad = _build_deg()(row)
    deg3d = jnp.concatenate(
        [deg_pad[:HALF], deg_pad[HALFP:HALFP + HALF]]).reshape(GRID, 1, MB)

    enc = _build_enc()
    upd = _build_upd()
    fin = _build_fin()
    agg = _build_agg()

    def pack_g(g):
        t = g.reshape(N, GWH // 32, 2, 16).transpose(0, 1, 3, 2)
        g32 = lax.bitcast_convert_type(t, _i32).reshape(N, GWH // 2)
        return jnp.concatenate(
            [g32, jnp.zeros((N, GW2 - GWH // 2), _i32)], axis=1)

    def run_agg(ga, gb):
        flat = agg(pack_g(ga), pack_g(gb), row, col, pe0, pe1)
        a0 = flat[:N * NF].reshape(N, NF)
        a1 = flat[NPADF:NPADF + N * NF].reshape(N, NF)
        return jnp.concatenate([a0, a1], axis=1)

    h, p, ga, gb = enc(x, enc_W, enc_b.reshape(1, H), Ws[0], bs[0])
    for l in range(1, 4):
        a = run_agg(ga, gb)
        h, p, ga, gb = upd(h, p, a, deg3d, Ws[l], bs[l])
    a = run_agg(ga, gb)
    return fin(h, p, a, deg3d, batch3d)
